# Initial kernel scaffold; baseline (speedup 1.0000x reference)
import jax
import jax.numpy as jnp
from jax.experimental import pallas as pl


def kernel(x, table):
    raise NotImplementedError("write your pallas kernel here")


# speedup vs baseline: 1.0266x; 1.0266x over previous
import functools

import jax
import jax.numpy as jnp
from jax import lax
from jax.experimental import pallas as pl
from jax.experimental.pallas import tpu as pltpu
from jax.experimental.pallas import tpu_sc as plsc

_VOCAB = 1000
_D = 1000
_N = 4096 * 50
_NC = 2
_NS = 16
_NW = _NC * _NS
_B_PER_W = _N // _NW
_CHUNK = 80
_NCHUNK = _B_PER_W // _CHUNK

_mesh = plsc.VectorSubcoreMesh(core_axis_name="c", subcore_axis_name="s")


@functools.partial(
    pl.kernel,
    mesh=_mesh,
    out_type=jax.ShapeDtypeStruct((_N, _D), jnp.float32),
    scratch_types=[
        pltpu.VMEM((_B_PER_W,), jnp.int32),
        pltpu.VMEM((_CHUNK, _D), jnp.float32),
        pltpu.SemaphoreType.DMA,
    ],
    compiler_params=pltpu.CompilerParams(use_tc_tiling_on_sc=False),
)
def _sc_gather(table_hbm, idx_hbm, out_hbm, idx_v, rows_v, sem):
    wid = lax.axis_index("s") * _NC + lax.axis_index("c")
    base = pl.multiple_of(wid * _B_PER_W, _B_PER_W)
    pltpu.sync_copy(idx_hbm.at[pl.ds(base, _B_PER_W)], idx_v)

    def body(k, carry):
        off = pl.multiple_of(k * _CHUNK, _CHUNK)
        idx_slice = idx_v.at[pl.ds(off, _CHUNK)]
        pltpu.async_copy(table_hbm.at[idx_slice], rows_v, sem).wait()
        pltpu.sync_copy(rows_v, out_hbm.at[pl.ds(base + off, _CHUNK)])
        return carry

    lax.fori_loop(0, _NCHUNK, body, 0)


def kernel(x, table):
    idx = x.reshape(-1).astype(jnp.int32)
    out = _sc_gather(table, idx)
    return out.reshape(x.shape + (table.shape[1],))


# speedup vs baseline: 1.5191x; 1.4797x over previous
import functools

import jax
import jax.numpy as jnp
from jax import lax
from jax.experimental import pallas as pl
from jax.experimental.pallas import tpu as pltpu
from jax.experimental.pallas import tpu_sc as plsc

_V = 1000
_D = 1000
_BATCH = 4096
_HIST = 50
_NVT = 125
_NBT = 32
_NC = 2
_NS = 16
_COLS = 64

_mesh = plsc.VectorSubcoreMesh(core_axis_name="c", subcore_axis_name="s")


@functools.partial(
    pl.kernel,
    mesh=_mesh,
    out_type=jax.ShapeDtypeStruct((_HIST, _NVT, _NBT, 8, 128), jnp.float32),
    scratch_types=[
        pltpu.VMEM((_COLS * _V,), jnp.float32),
        pltpu.VMEM((2, _BATCH), jnp.int32),
        pltpu.VMEM((2, 8, 8, 128), jnp.float32),
        pltpu.SemaphoreType.DMA,
        pltpu.SemaphoreType.DMA,
        pltpu.SemaphoreType.DMA,
    ],
    compiler_params=pltpu.CompilerParams(
        use_tc_tiling_on_sc=False, needs_layout_passes=False
    ),
)
def _sc_tgather(table_hbm, xt_hbm, out_hbm, tab_v, idx_v, stage_v,
                isem, asem, bsem):
    c = lax.axis_index("c")
    s = lax.axis_index("s")
    vt0 = s * _NVT // _NS
    nvt = (s + 1) * _NVT // _NS - vt0
    has8 = nvt == 8

    pltpu.sync_copy(table_hbm.at[pl.ds(vt0 * 8 * _V, _COLS * _V)], tab_v)

    def idx_start(l, slot):
        pltpu.async_copy(xt_hbm.at[l], idx_v.at[slot], isem)

    def idx_wait(slot):
        pltpu.make_async_copy(xt_hbm.at[0], idx_v.at[slot], isem).wait()

    def write_start(l, bt, oslot):
        pltpu.async_copy(
            stage_v.at[oslot, pl.ds(0, 7)],
            out_hbm.at[l, pl.ds(vt0, 7), bt],
            asem,
        )

        @pl.when(has8)
        def _():
            pltpu.async_copy(
                stage_v.at[oslot, 7], out_hbm.at[l, vt0 + 7, bt], bsem
            )

    def write_wait():
        pltpu.make_async_copy(
            stage_v.at[0, pl.ds(0, 7)], out_hbm.at[0, pl.ds(0, 7), 0], asem
        ).wait()

        @pl.when(has8)
        def _():
            pltpu.make_async_copy(
                stage_v.at[0, 7], out_hbm.at[0, 0, 0], bsem
            ).wait()

    idx_start(0, 0)

    def l_body(l, carry):
        lslot = l % 2
        idx_wait(lslot)

        @pl.when(l + 1 < _HIST)
        def _():
            idx_start(l + 1, 1 - lslot)

        def j_body(j, carry2):
            bt = j * _NC + c
            oslot = j % 2

            @pl.when((l > 0) | (j >= 2))
            def _():
                write_wait()

            rows = [
                idx_v.at[lslot][pl.ds(bt * 128 + g * 16, 16)] for g in range(8)
            ]

            def vt_body(vt, carry3):
                for vs in range(8):
                    colvec = jnp.full((16,), (vt * 8 + vs) * _V, dtype=jnp.int32)
                    for g in range(8):
                        vals = plsc.load_gather(tab_v, [rows[g] + colvec])
                        stage_v[oslot, vt, vs, pl.ds(g * 16, 16)] = vals
                return carry3

            lax.fori_loop(0, nvt, vt_body, 0)
            write_start(l, bt, oslot)
            return carry2

        lax.fori_loop(0, 16, j_body, 0)
        return carry

    lax.fori_loop(0, _HIST, l_body, 0)
    write_wait()
    write_wait()


def kernel(x, table):
    xt = jnp.transpose(x.astype(jnp.int32))
    tflat = jnp.transpose(table).reshape(-1)
    out5d = _sc_tgather(tflat, xt)
    return jnp.transpose(out5d, (2, 4, 0, 1, 3)).reshape(_BATCH, _HIST, _D)


# speedup vs baseline: 1.5243x; 1.0034x over previous
import functools

import jax
import jax.numpy as jnp
from jax import lax
from jax.experimental import pallas as pl
from jax.experimental.pallas import tpu as pltpu
from jax.experimental.pallas import tpu_sc as plsc

_V = 1000
_D = 1000
_BATCH = 4096
_HIST = 50
_NVT = 125
_NBT = 32
_NC = 2
_NS = 16
_COLS = 64

_mesh = plsc.VectorSubcoreMesh(core_axis_name="c", subcore_axis_name="s")


@functools.partial(
    pl.kernel,
    mesh=_mesh,
    out_type=jax.ShapeDtypeStruct((_HIST, _NVT, _NBT, 8, 128), jnp.float32),
    scratch_types=[
        pltpu.VMEM((_COLS * _V,), jnp.float32),
        pltpu.VMEM((2, _BATCH), jnp.int32),
        pltpu.VMEM((2, 8, 8, 128), jnp.float32),
        pltpu.SemaphoreType.DMA,
        pltpu.SemaphoreType.DMA,
        pltpu.SemaphoreType.DMA,
    ],
    compiler_params=pltpu.CompilerParams(
        use_tc_tiling_on_sc=False, needs_layout_passes=False
    ),
)
def _sc_tgather(table_hbm, xt_hbm, out_hbm, tab_v, idx_v, stage_v,
                isem, asem, bsem):
    c = lax.axis_index("c")
    s = lax.axis_index("s")
    vt0 = s * _NVT // _NS
    nvt = (s + 1) * _NVT // _NS - vt0
    has8 = nvt == 8

    pltpu.sync_copy(table_hbm.at[pl.ds(vt0 * 8 * _V, _COLS * _V)], tab_v)

    def idx_start(l, slot):
        pltpu.async_copy(xt_hbm.at[l], idx_v.at[slot], isem)

    def idx_wait(slot):
        pltpu.make_async_copy(xt_hbm.at[0], idx_v.at[slot], isem).wait()

    def write_start(l, bt, oslot):
        pltpu.async_copy(
            stage_v.at[oslot, pl.ds(0, 7)],
            out_hbm.at[l, pl.ds(vt0, 7), bt],
            asem,
        )

        @pl.when(has8)
        def _():
            pltpu.async_copy(
                stage_v.at[oslot, 7], out_hbm.at[l, vt0 + 7, bt], bsem
            )

    def write_wait():
        pltpu.make_async_copy(
            stage_v.at[0, pl.ds(0, 7)], out_hbm.at[0, pl.ds(0, 7), 0], asem
        ).wait()

        @pl.when(has8)
        def _():
            pltpu.make_async_copy(
                stage_v.at[0, 7], out_hbm.at[0, 0, 0], bsem
            ).wait()

    idx_start(0, 0)

    def l_body(l, carry):
        lslot = l % 2
        idx_wait(lslot)

        @pl.when(l + 1 < _HIST)
        def _():
            idx_start(l + 1, 1 - lslot)

        def j_body(j, carry2):
            bt = j * _NC + c
            oslot = j % 2

            @pl.when((l > 0) | (j >= 2))
            def _():
                write_wait()

            rows = [
                idx_v.at[lslot][pl.ds(bt * 128 + g * 16, 16)] for g in range(8)
            ]

            def do_vt(vt):
                for vs in range(8):
                    sub = tab_v.at[pl.ds((vt * 8 + vs) * _V, _V)]
                    for g in range(8):
                        vals = plsc.load_gather(sub, [rows[g]])
                        stage_v[oslot, vt, vs, pl.ds(g * 16, 16)] = vals

            for vt in range(7):
                do_vt(vt)

            @pl.when(has8)
            def _():
                do_vt(7)

            write_start(l, bt, oslot)
            return carry2

        lax.fori_loop(0, 16, j_body, 0)
        return carry

    lax.fori_loop(0, _HIST, l_body, 0)
    write_wait()
    write_wait()


def kernel(x, table):
    xt = jnp.transpose(x.astype(jnp.int32))
    tflat = jnp.transpose(table).reshape(-1)
    out5d = _sc_tgather(tflat, xt)
    return jnp.transpose(out5d, (2, 4, 0, 1, 3)).reshape(_BATCH, _HIST, _D)


# speedup vs baseline: 4.8605x; 3.1888x over previous
import functools

import jax
import jax.numpy as jnp
from jax import lax
from jax.experimental import pallas as pl
from jax.experimental.pallas import tpu as pltpu
from jax.experimental.pallas import tpu_sc as plsc

_V = 1000
_D = 1000
_BATCH = 4096
_HIST = 50
_NVT = 125
_NBT = 32
_NC = 2
_NS = 16
_COLS = 64

_mesh = plsc.VectorSubcoreMesh(core_axis_name="c", subcore_axis_name="s")


@functools.partial(
    pl.kernel,
    mesh=_mesh,
    out_type=jax.ShapeDtypeStruct((_HIST, _NVT, _NBT, 8, 128), jnp.float32),
    scratch_types=[
        pltpu.VMEM((_COLS * _V,), jnp.float32),
        pltpu.VMEM((2, _BATCH), jnp.int32),
        pltpu.VMEM((2, 8, 8, 128), jnp.float32),
        pltpu.SemaphoreType.DMA,
        pltpu.SemaphoreType.DMA,
        pltpu.SemaphoreType.DMA,
    ],
    compiler_params=pltpu.CompilerParams(
        use_tc_tiling_on_sc=False, needs_layout_passes=False
    ),
)
def _sc_tgather(table_hbm, xt_hbm, out_hbm, tab_v, idx_v, stage_v,
                isem, asem, bsem):
    c = lax.axis_index("c")
    s = lax.axis_index("s")
    vt0 = s * _NVT // _NS
    nvt = (s + 1) * _NVT // _NS - vt0
    has8 = nvt == 8

    pltpu.sync_copy(table_hbm.at[pl.ds(vt0 * 8 * _V, _COLS * _V)], tab_v)

    def idx_start(l, slot):
        pltpu.async_copy(xt_hbm.at[l], idx_v.at[slot], isem)

    def idx_wait(slot):
        pltpu.make_async_copy(xt_hbm.at[0], idx_v.at[slot], isem).wait()

    def write_start(l, bt, oslot):
        pltpu.async_copy(
            stage_v.at[oslot, pl.ds(0, 7)],
            out_hbm.at[l, pl.ds(vt0, 7), bt],
            asem,
        )

        @pl.when(has8)
        def _():
            pltpu.async_copy(
                stage_v.at[oslot, 7], out_hbm.at[l, vt0 + 7, bt], bsem
            )

    def write_wait():
        pltpu.make_async_copy(
            stage_v.at[0, pl.ds(0, 7)], out_hbm.at[0, pl.ds(0, 7), 0], asem
        ).wait()

        @pl.when(has8)
        def _():
            pltpu.make_async_copy(
                stage_v.at[0, 7], out_hbm.at[0, 0, 0], bsem
            ).wait()

    idx_start(0, 0)

    def l_body(l, carry):
        lslot = l % 2
        idx_wait(lslot)

        @pl.when(l + 1 < _HIST)
        def _():
            idx_start(l + 1, 1 - lslot)

        def j_body(j, carry2):
            bt = j * _NC + c
            oslot = j % 2

            @pl.when((l > 0) | (j >= 2))
            def _():
                write_wait()

            rows = [
                idx_v.at[lslot][pl.ds(bt * 128 + g * 16, 16)] for g in range(8)
            ]

            def do_vt(vt):
                for vs in range(8):
                    sub = tab_v.at[pl.ds((vt * 8 + vs) * _V, _V)]
                    vals = [plsc.load_gather(sub, [rows[g]]) for g in range(8)]
                    for g in range(8):
                        stage_v[oslot, vt, vs, pl.ds(g * 16, 16)] = vals[g]

            for vt in range(7):
                do_vt(vt)

            @pl.when(has8)
            def _():
                do_vt(7)

            write_start(l, bt, oslot)
            return carry2

        lax.fori_loop(0, 16, j_body, 0)
        return carry

    lax.fori_loop(0, _HIST, l_body, 0)
    write_wait()
    write_wait()


def kernel(x, table):
    xt = jnp.transpose(x.astype(jnp.int32))
    tflat = jnp.transpose(table).reshape(-1)
    out5d = _sc_tgather(tflat, xt)
    return jnp.transpose(out5d, (2, 4, 0, 1, 3)).reshape(_BATCH, _HIST, _D)
